# baseline (speedup 1.0000x reference)
import numpy as np
import jax
import jax.numpy as jnp
from jax.experimental import pallas as pl
from jax.experimental.pallas import tpu as pltpu

B_TOTAL = 4096
BT = 64
N_TILES = B_TOTAL // BT

_CT_TAPS = (((1, 1), (3, 0)), ((0, 2), (2, 1)))


def _sel_conv(n_in, n_out, kw):
    s = np.zeros((n_in, n_out), np.float32)
    for ow in range(n_out):
        j = 2 * ow + kw - 1
        if 0 <= j < n_in:
            s[j, ow] = 1.0
    return s


def _sel_convt(n_in, n_out, kw):
    s = np.zeros((n_in, n_out), np.float32)
    for iw in range(n_in):
        k = np.arange(n_out) - 2 * iw + 1
        for ow in range(n_out):
            if k[ow] == kw:
                s[iw, ow] = 1.0
    return s


def _dot3(a, b):
    ah = a.astype(jnp.bfloat16)
    al = (a - ah.astype(jnp.float32)).astype(jnp.bfloat16)
    bh = b.astype(jnp.bfloat16)
    bl = (b - bh.astype(jnp.float32)).astype(jnp.bfloat16)
    return (jnp.dot(ah, bh, preferred_element_type=jnp.float32)
            + jnp.dot(ah, bl, preferred_element_type=jnp.float32)
            + jnp.dot(al, bh, preferred_element_type=jnp.float32))


def _dot16(a, b):
    return jnp.dot(a.astype(jnp.bfloat16), b.astype(jnp.bfloat16),
                   preferred_element_type=jnp.float32)


def _fwd_kernel(x_ref, m1_ref, b1_ref, m2_ref, b2_ref, embt_ref,
                m3_ref, b3_ref, m4_ref, b4_ref,
                xr_ref, loss_ref, perp_ref, enc_ref,
                loss_acc, cnt_acc):
    i = pl.program_id(0)

    @pl.when(i == 0)
    def _init():
        loss_acc[...] = jnp.zeros((1, 1), jnp.float32)
        cnt_acc[...] = jnp.zeros((1, 10), jnp.float32)

    x = x_ref[...]

    xph = jnp.pad(x, ((0, 0), (1, 1), (0, 0)))
    pr = xph.reshape(BT, 15, 2, 28)
    p1 = (pr[:, :, 0, :], pr[:, :, 1, :])
    y1 = jnp.broadcast_to(b1_ref[...], (BT * 14, 448))
    for kh in range(4):
        v = p1[kh % 2][:, kh // 2:kh // 2 + 14, :].reshape(BT * 14, 28)
        y1 = y1 + jnp.dot(v, m1_ref[kh], preferred_element_type=jnp.float32)
    y1 = jnp.maximum(y1, 0.0).reshape(BT, 14, 448)

    y1p = jnp.pad(y1, ((0, 0), (1, 1), (0, 0)))
    pr2 = y1p.reshape(BT, 8, 2, 448)
    p2 = (pr2[:, :, 0, :], pr2[:, :, 1, :])
    z2 = jnp.broadcast_to(b2_ref[...], (BT * 7, 448))
    for kh in range(4):
        v = p2[kh % 2][:, kh // 2:kh // 2 + 7, :].reshape(BT * 7, 448)
        z2 = z2 + _dot3(v, m2_ref[kh])
    z2 = jnp.maximum(z2, 0.0).reshape(BT, 7, 448)

    embt = embt_ref[...]
    dots = jnp.zeros((BT, 10), jnp.float32)
    for oh in range(7):
        dots = dots + jnp.dot(z2[:, oh, :].reshape(BT, 448),
                              embt[oh * 448:(oh + 1) * 448, :],
                              preferred_element_type=jnp.float32)
    znorm = jnp.sum(jnp.sum(z2 * z2, axis=2), axis=1, keepdims=True)
    enorm = jnp.sum(embt * embt, axis=0)[None, :]
    d = znorm + enorm - 2.0 * dots
    idx = jnp.argmin(d, axis=1)
    enc = (jax.lax.broadcasted_iota(jnp.int32, (BT, 10), 1)
           == idx[:, None]).astype(jnp.float32)
    enc_ref[...] = enc
    loss_acc[...] += jnp.sum(jnp.min(d, axis=1)).reshape(1, 1)
    cnt_acc[...] += jnp.sum(enc, axis=0, keepdims=True)

    z2p = jnp.pad(z2, ((0, 0), (1, 1), (0, 0)))
    hplanes = []
    for r in range(2):
        a = jnp.broadcast_to(b3_ref[...], (BT * 7, 448))
        for kh, ro in _CT_TAPS[r]:
            v = z2p[:, ro:ro + 7, :].reshape(BT * 7, 448)
            a = a + _dot16(v, m3_ref[kh])
        hplanes.append(jnp.maximum(a, 0.0).reshape(BT, 7, 448))
    h = jnp.stack(hplanes, axis=2).reshape(BT, 14, 448)

    hp = jnp.pad(h, ((0, 0), (1, 1), (0, 0)))
    for r2 in range(2):
        a = jnp.zeros((BT * 14, 28), jnp.float32)
        for kh, ro in _CT_TAPS[r2]:
            v = hp[:, ro:ro + 14, :].reshape(BT * 14, 448)
            a = a + _dot16(v, m4_ref[kh])
        xr_ref[:, r2, :, :] = jax.nn.sigmoid(a + b4_ref[...]) \
            .reshape(BT, 14, 28)

    @pl.when(i == N_TILES - 1)
    def _fin():
        loss_ref[...] = loss_acc[...] * (1.25 / (B_TOTAL * 3136.0))
        p = cnt_acc[...] / B_TOTAL
        perp_ref[...] = jnp.exp(-jnp.sum(p * jnp.log(p + 1e-10))).reshape(1, 1)


def kernel(x, conv1_w, conv1_b, conv2_w, conv2_b, emb,
           dec1_w, dec1_b, dec2_w, dec2_b):
    x3 = x.reshape(B_TOTAL, 28, 28)
    embt = emb.T

    m1 = jnp.stack([
        sum(jnp.asarray(_sel_conv(28, 14, kw))[:, :, None]
            * conv1_w[:, 0, kh, kw][None, None, :] for kw in range(4))
        .reshape(28, 448) for kh in range(4)])
    m2 = jnp.stack([
        sum(jnp.asarray(_sel_conv(14, 7, kw))[:, None, :, None]
            * conv2_w[:, :, kh, kw].T[None, :, None, :] for kw in range(4))
        .reshape(448, 448) for kh in range(4)])
    m3 = jnp.stack([
        sum(jnp.asarray(_sel_convt(7, 14, kw))[:, None, :, None]
            * dec1_w[:, :, kh, kw][None, :, None, :] for kw in range(4))
        .reshape(448, 448) for kh in range(4)])
    m4 = jnp.stack([
        sum(jnp.asarray(_sel_convt(14, 28, kw))[:, None, :]
            * dec2_w[:, 0, kh, kw][None, :, None] for kw in range(4))
        .reshape(448, 28) for kh in range(4)])
    b1t = jnp.tile(conv1_b, 14)[None, :]
    b2t = jnp.tile(conv2_b, 7)[None, :]
    b3t = jnp.tile(dec1_b, 14)[None, :]

    grid = (N_TILES,)
    xr, loss, perp, enc = pl.pallas_call(
        _fwd_kernel,
        grid=grid,
        in_specs=[
            pl.BlockSpec((BT, 28, 28), lambda i: (i, 0, 0)),
            pl.BlockSpec((4, 28, 448), lambda i: (0, 0, 0)),
            pl.BlockSpec((1, 448), lambda i: (0, 0)),
            pl.BlockSpec((4, 448, 448), lambda i: (0, 0, 0)),
            pl.BlockSpec((1, 448), lambda i: (0, 0)),
            pl.BlockSpec((3136, 10), lambda i: (0, 0)),
            pl.BlockSpec((4, 448, 448), lambda i: (0, 0, 0)),
            pl.BlockSpec((1, 448), lambda i: (0, 0)),
            pl.BlockSpec((4, 448, 28), lambda i: (0, 0, 0)),
            pl.BlockSpec((1, 1), lambda i: (0, 0)),
        ],
        out_specs=[
            pl.BlockSpec((BT, 2, 14, 28), lambda i: (i, 0, 0, 0)),
            pl.BlockSpec((1, 1), lambda i: (0, 0)),
            pl.BlockSpec((1, 1), lambda i: (0, 0)),
            pl.BlockSpec((BT, 10), lambda i: (i, 0)),
        ],
        out_shape=[
            jax.ShapeDtypeStruct((B_TOTAL, 2, 14, 28), jnp.float32),
            jax.ShapeDtypeStruct((1, 1), jnp.float32),
            jax.ShapeDtypeStruct((1, 1), jnp.float32),
            jax.ShapeDtypeStruct((B_TOTAL, 10), jnp.float32),
        ],
        scratch_shapes=[
            pltpu.VMEM((1, 1), jnp.float32),
            pltpu.VMEM((1, 10), jnp.float32),
        ],
        compiler_params=pltpu.CompilerParams(
            dimension_semantics=("arbitrary",)),
    )(x3, m1, b1t, m2, b2t, embt, m3, b3t, m4, dec2_b[None, :])

    xrec = xr.transpose(0, 2, 1, 3).reshape(B_TOTAL, 1, 28, 28)
    return (xrec, loss[0, 0], perp[0, 0], enc)


# speedup vs baseline: 1.2189x; 1.2189x over previous
import numpy as np
import jax
import jax.numpy as jnp
from jax.experimental import pallas as pl
from jax.experimental.pallas import tpu as pltpu

B_TOTAL = 4096
BT = 64
N_TILES = B_TOTAL // BT

_CT_TAPS = (((1, 1), (3, 0)), ((0, 2), (2, 1)))


def _sel_conv(n_in, n_out, kw):
    s = np.zeros((n_in, n_out), np.float32)
    for ow in range(n_out):
        j = 2 * ow + kw - 1
        if 0 <= j < n_in:
            s[j, ow] = 1.0
    return s


def _sel_convt(n_in, n_out, kw):
    s = np.zeros((n_in, n_out), np.float32)
    for iw in range(n_in):
        k = np.arange(n_out) - 2 * iw + 1
        for ow in range(n_out):
            if k[ow] == kw:
                s[iw, ow] = 1.0
    return s


def _dot(a, b):
    return jnp.dot(a, b, preferred_element_type=jnp.float32)


def _split(a):
    hi = a.astype(jnp.bfloat16)
    lo = (a - hi.astype(jnp.float32)).astype(jnp.bfloat16)
    return hi, lo


def _fwd_kernel(x_ref, m1_ref, b1_ref, m2h_ref, m2l_ref, b2_ref, embt_ref,
                m3_ref, b3_ref, m4_ref, b4_ref,
                xr_ref, loss_ref, perp_ref, enc_ref,
                loss_acc, cnt_acc):
    i = pl.program_id(0)

    @pl.when(i == 0)
    def _init():
        loss_acc[...] = jnp.zeros((1, 1), jnp.float32)
        cnt_acc[...] = jnp.zeros((1, 10), jnp.float32)

    x = x_ref[...]

    xph = jnp.pad(x, ((0, 0), (1, 1), (0, 0)))
    pr = xph.reshape(BT, 15, 2, 28)
    p1 = (pr[:, :, 0, :], pr[:, :, 1, :])
    y1 = jnp.broadcast_to(b1_ref[...], (BT * 14, 448))
    for kh in range(4):
        v = p1[kh % 2][:, kh // 2:kh // 2 + 14, :].reshape(BT * 14, 28)
        y1 = y1 + jnp.dot(v, m1_ref[kh], preferred_element_type=jnp.float32)
    y1 = jnp.maximum(y1, 0.0).reshape(BT, 14, 448)

    y1h, y1l = _split(jnp.pad(y1, ((0, 0), (1, 1), (0, 0))))
    p2h = y1h.reshape(BT, 8, 2, 448)
    p2l = y1l.reshape(BT, 8, 2, 448)
    z2 = jnp.broadcast_to(b2_ref[...], (BT * 7, 448))
    for kh in range(4):
        vh = p2h[:, :, kh % 2, :][:, kh // 2:kh // 2 + 7, :] \
            .reshape(BT * 7, 448)
        vl = p2l[:, :, kh % 2, :][:, kh // 2:kh // 2 + 7, :] \
            .reshape(BT * 7, 448)
        z2 = (z2 + _dot(vh, m2h_ref[kh]) + _dot(vh, m2l_ref[kh])
              + _dot(vl, m2h_ref[kh]))
    z2 = jnp.maximum(z2, 0.0).reshape(BT, 7, 448)

    embt = embt_ref[...]
    dots = jnp.zeros((BT, 10), jnp.float32)
    for oh in range(7):
        dots = dots + jnp.dot(z2[:, oh, :].reshape(BT, 448),
                              embt[oh * 448:(oh + 1) * 448, :],
                              preferred_element_type=jnp.float32)
    znorm = jnp.sum(jnp.sum(z2 * z2, axis=2), axis=1, keepdims=True)
    enorm = jnp.sum(embt * embt, axis=0)[None, :]
    d = znorm + enorm - 2.0 * dots
    idx = jnp.argmin(d, axis=1)
    enc = (jax.lax.broadcasted_iota(jnp.int32, (BT, 10), 1)
           == idx[:, None]).astype(jnp.float32)
    enc_ref[...] = enc
    loss_acc[...] += jnp.sum(jnp.min(d, axis=1)).reshape(1, 1)
    cnt_acc[...] += jnp.sum(enc, axis=0, keepdims=True)

    z2p = jnp.pad(z2, ((0, 0), (1, 1), (0, 0))) \
        .astype(jnp.bfloat16)
    hplanes = []
    for r in range(2):
        a = jnp.broadcast_to(b3_ref[...], (BT * 7, 448))
        for kh, ro in _CT_TAPS[r]:
            v = z2p[:, ro:ro + 7, :].reshape(BT * 7, 448)
            a = a + _dot(v, m3_ref[kh])
        hplanes.append(jnp.maximum(a, 0.0).reshape(BT, 7, 448))
    h = jnp.stack(hplanes, axis=2).reshape(BT, 14, 448)

    hp = jnp.pad(h, ((0, 0), (1, 1), (0, 0))) \
        .astype(jnp.bfloat16)
    for r2 in range(2):
        a = jnp.zeros((BT * 14, 28), jnp.float32)
        for kh, ro in _CT_TAPS[r2]:
            v = hp[:, ro:ro + 14, :].reshape(BT * 14, 448)
            a = a + _dot(v, m4_ref[kh])
        xr_ref[:, r2, :, :] = jax.nn.sigmoid(a + b4_ref[...]) \
            .reshape(BT, 14, 28)

    @pl.when(i == N_TILES - 1)
    def _fin():
        loss_ref[...] = loss_acc[...] * (1.25 / (B_TOTAL * 3136.0))
        p = cnt_acc[...] / B_TOTAL
        perp_ref[...] = jnp.exp(-jnp.sum(p * jnp.log(p + 1e-10))).reshape(1, 1)


def kernel(x, conv1_w, conv1_b, conv2_w, conv2_b, emb,
           dec1_w, dec1_b, dec2_w, dec2_b):
    x3 = x.reshape(B_TOTAL, 28, 28)
    embt = emb.T

    m1 = jnp.stack([
        sum(jnp.asarray(_sel_conv(28, 14, kw))[:, :, None]
            * conv1_w[:, 0, kh, kw][None, None, :] for kw in range(4))
        .reshape(28, 448) for kh in range(4)])
    m2 = jnp.stack([
        sum(jnp.asarray(_sel_conv(14, 7, kw))[:, None, :, None]
            * conv2_w[:, :, kh, kw].T[None, :, None, :] for kw in range(4))
        .reshape(448, 448) for kh in range(4)])
    m3 = jnp.stack([
        sum(jnp.asarray(_sel_convt(7, 14, kw))[:, None, :, None]
            * dec1_w[:, :, kh, kw][None, :, None, :] for kw in range(4))
        .reshape(448, 448) for kh in range(4)])
    m4 = jnp.stack([
        sum(jnp.asarray(_sel_convt(14, 28, kw))[:, None, :]
            * dec2_w[:, 0, kh, kw][None, :, None] for kw in range(4))
        .reshape(448, 28) for kh in range(4)])
    b1t = jnp.tile(conv1_b, 14)[None, :]
    b2t = jnp.tile(conv2_b, 7)[None, :]
    b3t = jnp.tile(dec1_b, 14)[None, :]
    m2h = m2.astype(jnp.bfloat16)
    m2l = (m2 - m2h.astype(jnp.float32)).astype(jnp.bfloat16)
    m3 = m3.astype(jnp.bfloat16)
    m4 = m4.astype(jnp.bfloat16)

    grid = (N_TILES,)
    xr, loss, perp, enc = pl.pallas_call(
        _fwd_kernel,
        grid=grid,
        in_specs=[
            pl.BlockSpec((BT, 28, 28), lambda i: (i, 0, 0)),
            pl.BlockSpec((4, 28, 448), lambda i: (0, 0, 0)),
            pl.BlockSpec((1, 448), lambda i: (0, 0)),
            pl.BlockSpec((4, 448, 448), lambda i: (0, 0, 0)),
            pl.BlockSpec((4, 448, 448), lambda i: (0, 0, 0)),
            pl.BlockSpec((1, 448), lambda i: (0, 0)),
            pl.BlockSpec((3136, 10), lambda i: (0, 0)),
            pl.BlockSpec((4, 448, 448), lambda i: (0, 0, 0)),
            pl.BlockSpec((1, 448), lambda i: (0, 0)),
            pl.BlockSpec((4, 448, 28), lambda i: (0, 0, 0)),
            pl.BlockSpec((1, 1), lambda i: (0, 0)),
        ],
        out_specs=[
            pl.BlockSpec((BT, 2, 14, 28), lambda i: (i, 0, 0, 0)),
            pl.BlockSpec((1, 1), lambda i: (0, 0)),
            pl.BlockSpec((1, 1), lambda i: (0, 0)),
            pl.BlockSpec((BT, 10), lambda i: (i, 0)),
        ],
        out_shape=[
            jax.ShapeDtypeStruct((B_TOTAL, 2, 14, 28), jnp.float32),
            jax.ShapeDtypeStruct((1, 1), jnp.float32),
            jax.ShapeDtypeStruct((1, 1), jnp.float32),
            jax.ShapeDtypeStruct((B_TOTAL, 10), jnp.float32),
        ],
        scratch_shapes=[
            pltpu.VMEM((1, 1), jnp.float32),
            pltpu.VMEM((1, 10), jnp.float32),
        ],
        compiler_params=pltpu.CompilerParams(
            dimension_semantics=("arbitrary",)),
    )(x3, m1, b1t, m2h, m2l, b2t, embt, m3, b3t, m4, dec2_b[None, :])

    xrec = xr.transpose(0, 2, 1, 3).reshape(B_TOTAL, 1, 28, 28)
    return (xrec, loss[0, 0], perp[0, 0], enc)


# speedup vs baseline: 1.6203x; 1.3292x over previous
import numpy as np
import jax
import jax.numpy as jnp
from jax.experimental import pallas as pl
from jax.experimental.pallas import tpu as pltpu

B_TOTAL = 4096
BT = 128
N_TILES = B_TOTAL // BT

_CT_TAPS = (((1, 1), (3, 0)), ((0, 2), (2, 1)))


def _sel_conv(n_in, n_out, kw):
    s = np.zeros((n_in, n_out), np.float32)
    for ow in range(n_out):
        j = 2 * ow + kw - 1
        if 0 <= j < n_in:
            s[j, ow] = 1.0
    return s


def _sel_convt(n_in, n_out, kw):
    s = np.zeros((n_in, n_out), np.float32)
    for iw in range(n_in):
        k = np.arange(n_out) - 2 * iw + 1
        for ow in range(n_out):
            if k[ow] == kw:
                s[iw, ow] = 1.0
    return s


def _dot(a, b):
    return jnp.dot(a, b, preferred_element_type=jnp.float32)


def _fwd_kernel(x_ref, m1_ref, b1_ref, m2_ref, b2_ref, embt_ref,
                m3_ref, b3_ref, m4_ref, b4_ref,
                xr_ref, loss_ref, perp_ref, enc_ref,
                loss_acc, cnt_acc):
    i = pl.program_id(0)

    @pl.when(i == 0)
    def _init():
        loss_acc[...] = jnp.zeros((1, 1), jnp.float32)
        cnt_acc[...] = jnp.zeros((1, 10), jnp.float32)

    x = x_ref[...]

    xph = jnp.pad(x, ((0, 0), (1, 1), (0, 0)))
    pr = xph.reshape(BT, 15, 2, 28)
    p1 = (pr[:, :, 0, :], pr[:, :, 1, :])
    y1 = jnp.broadcast_to(b1_ref[...], (BT * 14, 448))
    for kh in range(4):
        v = p1[kh % 2][:, kh // 2:kh // 2 + 14, :].reshape(BT * 14, 28)
        y1 = y1 + jnp.dot(v, m1_ref[kh], preferred_element_type=jnp.float32)
    y1 = jnp.maximum(y1, 0.0).reshape(BT, 14, 448)

    y1p = jnp.pad(y1, ((0, 0), (1, 1), (0, 0)))
    pr2 = y1p.reshape(BT, 8, 2, 448)
    p2 = (pr2[:, :, 0, :], pr2[:, :, 1, :])
    z2 = jnp.broadcast_to(b2_ref[...], (BT * 7, 448))
    for kh in range(4):
        v = p2[kh % 2][:, kh // 2:kh // 2 + 7, :].reshape(BT * 7, 448)
        z2 = z2 + _dot(v, m2_ref[kh])
    z2 = jnp.maximum(z2, 0.0).reshape(BT, 7, 448)

    embt = embt_ref[...]
    dots = jnp.zeros((BT, 10), jnp.float32)
    for oh in range(7):
        dots = dots + jnp.dot(z2[:, oh, :].reshape(BT, 448),
                              embt[oh * 448:(oh + 1) * 448, :],
                              preferred_element_type=jnp.float32)
    znorm = jnp.sum(jnp.sum(z2 * z2, axis=2), axis=1, keepdims=True)
    enorm = jnp.sum(embt * embt, axis=0)[None, :]
    d = znorm + enorm - 2.0 * dots
    idx = jnp.argmin(d, axis=1)
    enc = (jax.lax.broadcasted_iota(jnp.int32, (BT, 10), 1)
           == idx[:, None]).astype(jnp.float32)
    enc_ref[...] = enc
    loss_acc[...] += jnp.sum(jnp.min(d, axis=1)).reshape(1, 1)
    cnt_acc[...] += jnp.sum(enc, axis=0, keepdims=True)

    z2p = jnp.pad(z2, ((0, 0), (1, 1), (0, 0)))
    hplanes = []
    for r in range(2):
        a = jnp.broadcast_to(b3_ref[...], (BT * 7, 448))
        for kh, ro in _CT_TAPS[r]:
            v = z2p[:, ro:ro + 7, :].reshape(BT * 7, 448)
            a = a + _dot(v, m3_ref[kh])
        hplanes.append(jnp.maximum(a, 0.0).reshape(BT, 7, 448))
    h = jnp.stack(hplanes, axis=2).reshape(BT, 14, 448)

    hp = jnp.pad(h, ((0, 0), (1, 1), (0, 0)))
    for r2 in range(2):
        a = jnp.zeros((BT * 14, 28), jnp.float32)
        for kh, ro in _CT_TAPS[r2]:
            v = hp[:, ro:ro + 14, :].reshape(BT * 14, 448)
            a = a + _dot(v, m4_ref[kh])
        xr_ref[:, r2, :, :] = jax.nn.sigmoid(a + b4_ref[...]) \
            .reshape(BT, 14, 28)

    @pl.when(i == N_TILES - 1)
    def _fin():
        loss_ref[...] = loss_acc[...] * (1.25 / (B_TOTAL * 3136.0))
        p = cnt_acc[...] / B_TOTAL
        perp_ref[...] = jnp.exp(-jnp.sum(p * jnp.log(p + 1e-10))).reshape(1, 1)


def kernel(x, conv1_w, conv1_b, conv2_w, conv2_b, emb,
           dec1_w, dec1_b, dec2_w, dec2_b):
    x3 = x.reshape(B_TOTAL, 28, 28)
    embt = emb.T

    m1 = jnp.stack([
        sum(jnp.asarray(_sel_conv(28, 14, kw))[:, :, None]
            * conv1_w[:, 0, kh, kw][None, None, :] for kw in range(4))
        .reshape(28, 448) for kh in range(4)])
    m2 = jnp.stack([
        sum(jnp.asarray(_sel_conv(14, 7, kw))[:, None, :, None]
            * conv2_w[:, :, kh, kw].T[None, :, None, :] for kw in range(4))
        .reshape(448, 448) for kh in range(4)])
    m3 = jnp.stack([
        sum(jnp.asarray(_sel_convt(7, 14, kw))[:, None, :, None]
            * dec1_w[:, :, kh, kw][None, :, None, :] for kw in range(4))
        .reshape(448, 448) for kh in range(4)])
    m4 = jnp.stack([
        sum(jnp.asarray(_sel_convt(14, 28, kw))[:, None, :]
            * dec2_w[:, 0, kh, kw][None, :, None] for kw in range(4))
        .reshape(448, 28) for kh in range(4)])
    b1t = jnp.tile(conv1_b, 14)[None, :]
    b2t = jnp.tile(conv2_b, 7)[None, :]
    b3t = jnp.tile(dec1_b, 14)[None, :]

    grid = (N_TILES,)
    xr, loss, perp, enc = pl.pallas_call(
        _fwd_kernel,
        grid=grid,
        in_specs=[
            pl.BlockSpec((BT, 28, 28), lambda i: (i, 0, 0)),
            pl.BlockSpec((4, 28, 448), lambda i: (0, 0, 0)),
            pl.BlockSpec((1, 448), lambda i: (0, 0)),
            pl.BlockSpec((4, 448, 448), lambda i: (0, 0, 0)),
            pl.BlockSpec((1, 448), lambda i: (0, 0)),
            pl.BlockSpec((3136, 10), lambda i: (0, 0)),
            pl.BlockSpec((4, 448, 448), lambda i: (0, 0, 0)),
            pl.BlockSpec((1, 448), lambda i: (0, 0)),
            pl.BlockSpec((4, 448, 28), lambda i: (0, 0, 0)),
            pl.BlockSpec((1, 1), lambda i: (0, 0)),
        ],
        out_specs=[
            pl.BlockSpec((BT, 2, 14, 28), lambda i: (i, 0, 0, 0)),
            pl.BlockSpec((1, 1), lambda i: (0, 0)),
            pl.BlockSpec((1, 1), lambda i: (0, 0)),
            pl.BlockSpec((BT, 10), lambda i: (i, 0)),
        ],
        out_shape=[
            jax.ShapeDtypeStruct((B_TOTAL, 2, 14, 28), jnp.float32),
            jax.ShapeDtypeStruct((1, 1), jnp.float32),
            jax.ShapeDtypeStruct((1, 1), jnp.float32),
            jax.ShapeDtypeStruct((B_TOTAL, 10), jnp.float32),
        ],
        scratch_shapes=[
            pltpu.VMEM((1, 1), jnp.float32),
            pltpu.VMEM((1, 10), jnp.float32),
        ],
        compiler_params=pltpu.CompilerParams(
            dimension_semantics=("arbitrary",)),
    )(x3, m1, b1t, m2, b2t, embt, m3, b3t, m4, dec2_b[None, :])

    xrec = xr.transpose(0, 2, 1, 3).reshape(B_TOTAL, 1, 28, 28)
    return (xrec, loss[0, 0], perp[0, 0], enc)


# speedup vs baseline: 1.6213x; 1.0006x over previous
import numpy as np
import jax
import jax.numpy as jnp
from jax.experimental import pallas as pl
from jax.experimental.pallas import tpu as pltpu

B_TOTAL = 4096
BT = 128
N_TILES = B_TOTAL // BT

_CT_TAPS = (((1, 1), (3, 0)), ((0, 2), (2, 1)))


def _sel_conv(n_in, n_out, kw):
    s = np.zeros((n_in, n_out), np.float32)
    for ow in range(n_out):
        j = 2 * ow + kw - 1
        if 0 <= j < n_in:
            s[j, ow] = 1.0
    return s


def _sel_convt(n_in, n_out, kw):
    s = np.zeros((n_in, n_out), np.float32)
    for iw in range(n_in):
        k = np.arange(n_out) - 2 * iw + 1
        for ow in range(n_out):
            if k[ow] == kw:
                s[iw, ow] = 1.0
    return s


def _dot(a, b):
    return jnp.dot(a, b, preferred_element_type=jnp.float32)


def _fwd_kernel(x_ref, m1_ref, b1_ref, m2_ref, b2_ref, embt_ref,
                m3_ref, b3_ref, m4_ref, b4_ref,
                xr_ref, lp_ref, cp_ref, enc_ref):
    x = x_ref[...]

    xph = jnp.pad(x, ((0, 0), (1, 1), (0, 0)))
    pr = xph.reshape(BT, 15, 2, 28)
    p1 = (pr[:, :, 0, :], pr[:, :, 1, :])
    y1 = jnp.broadcast_to(b1_ref[...], (BT * 14, 448))
    for kh in range(4):
        v = p1[kh % 2][:, kh // 2:kh // 2 + 14, :].reshape(BT * 14, 28)
        y1 = y1 + jnp.dot(v, m1_ref[kh], preferred_element_type=jnp.float32)
    y1 = jnp.maximum(y1, 0.0).reshape(BT, 14, 448)

    y1p = jnp.pad(y1, ((0, 0), (1, 1), (0, 0)))
    pr2 = y1p.reshape(BT, 8, 2, 448)
    p2 = (pr2[:, :, 0, :], pr2[:, :, 1, :])
    z2 = jnp.broadcast_to(b2_ref[...], (BT * 7, 448))
    for kh in range(4):
        v = p2[kh % 2][:, kh // 2:kh // 2 + 7, :].reshape(BT * 7, 448)
        z2 = z2 + _dot(v, m2_ref[kh])
    z2 = jnp.maximum(z2, 0.0).reshape(BT, 7, 448)

    embt = embt_ref[...]
    dots = jnp.zeros((BT, 10), jnp.float32)
    for oh in range(7):
        dots = dots + jnp.dot(z2[:, oh, :].reshape(BT, 448),
                              embt[oh * 448:(oh + 1) * 448, :],
                              preferred_element_type=jnp.float32)
    znorm = jnp.sum(jnp.sum(z2 * z2, axis=2), axis=1, keepdims=True)
    enorm = jnp.sum(embt * embt, axis=0)[None, :]
    d = znorm + enorm - 2.0 * dots
    idx = jnp.argmin(d, axis=1)
    enc = (jax.lax.broadcasted_iota(jnp.int32, (BT, 10), 1)
           == idx[:, None]).astype(jnp.float32)
    enc_ref[...] = enc
    lp_ref[...] = jnp.sum(jnp.min(d, axis=1)).reshape(1, 1, 1)
    cp_ref[...] = jnp.sum(enc, axis=0).reshape(1, 1, 10)

    z2p = jnp.pad(z2, ((0, 0), (1, 1), (0, 0)))
    hplanes = []
    for r in range(2):
        a = jnp.broadcast_to(b3_ref[...], (BT * 7, 448))
        for kh, ro in _CT_TAPS[r]:
            v = z2p[:, ro:ro + 7, :].reshape(BT * 7, 448)
            a = a + _dot(v, m3_ref[kh])
        hplanes.append(jnp.maximum(a, 0.0).reshape(BT, 7, 448))
    h = jnp.stack(hplanes, axis=2).reshape(BT, 14, 448)

    hp = jnp.pad(h, ((0, 0), (1, 1), (0, 0)))
    for r2 in range(2):
        a = jnp.zeros((BT * 14, 28), jnp.float32)
        for kh, ro in _CT_TAPS[r2]:
            v = hp[:, ro:ro + 14, :].reshape(BT * 14, 448)
            a = a + _dot(v, m4_ref[kh])
        xr_ref[:, r2, :, :] = jax.nn.sigmoid(a + b4_ref[...]) \
            .reshape(BT, 14, 28)


def _fin_kernel(lp_ref, cp_ref, loss_ref, perp_ref):
    loss_ref[...] = jnp.sum(lp_ref[...]).reshape(1, 1) \
        * (1.25 / (B_TOTAL * 3136.0))
    p = jnp.sum(cp_ref[...], axis=(0, 1)).reshape(1, 10) / B_TOTAL
    perp_ref[...] = jnp.exp(-jnp.sum(p * jnp.log(p + 1e-10))).reshape(1, 1)


def kernel(x, conv1_w, conv1_b, conv2_w, conv2_b, emb,
           dec1_w, dec1_b, dec2_w, dec2_b):
    x3 = x.reshape(B_TOTAL, 28, 28)
    embt = emb.T

    m1 = jnp.stack([
        sum(jnp.asarray(_sel_conv(28, 14, kw))[:, :, None]
            * conv1_w[:, 0, kh, kw][None, None, :] for kw in range(4))
        .reshape(28, 448) for kh in range(4)])
    m2 = jnp.stack([
        sum(jnp.asarray(_sel_conv(14, 7, kw))[:, None, :, None]
            * conv2_w[:, :, kh, kw].T[None, :, None, :] for kw in range(4))
        .reshape(448, 448) for kh in range(4)])
    m3 = jnp.stack([
        sum(jnp.asarray(_sel_convt(7, 14, kw))[:, None, :, None]
            * dec1_w[:, :, kh, kw][None, :, None, :] for kw in range(4))
        .reshape(448, 448) for kh in range(4)])
    m4 = jnp.stack([
        sum(jnp.asarray(_sel_convt(14, 28, kw))[:, None, :]
            * dec2_w[:, 0, kh, kw][None, :, None] for kw in range(4))
        .reshape(448, 28) for kh in range(4)])
    b1t = jnp.tile(conv1_b, 14)[None, :]
    b2t = jnp.tile(conv2_b, 7)[None, :]
    b3t = jnp.tile(dec1_b, 14)[None, :]

    grid = (N_TILES,)
    xr, lp, cp, enc = pl.pallas_call(
        _fwd_kernel,
        grid=grid,
        in_specs=[
            pl.BlockSpec((BT, 28, 28), lambda i: (i, 0, 0)),
            pl.BlockSpec((4, 28, 448), lambda i: (0, 0, 0)),
            pl.BlockSpec((1, 448), lambda i: (0, 0)),
            pl.BlockSpec((4, 448, 448), lambda i: (0, 0, 0)),
            pl.BlockSpec((1, 448), lambda i: (0, 0)),
            pl.BlockSpec((3136, 10), lambda i: (0, 0)),
            pl.BlockSpec((4, 448, 448), lambda i: (0, 0, 0)),
            pl.BlockSpec((1, 448), lambda i: (0, 0)),
            pl.BlockSpec((4, 448, 28), lambda i: (0, 0, 0)),
            pl.BlockSpec((1, 1), lambda i: (0, 0)),
        ],
        out_specs=[
            pl.BlockSpec((BT, 2, 14, 28), lambda i: (i, 0, 0, 0)),
            pl.BlockSpec((1, 1, 1), lambda i: (i, 0, 0)),
            pl.BlockSpec((1, 1, 10), lambda i: (i, 0, 0)),
            pl.BlockSpec((BT, 10), lambda i: (i, 0)),
        ],
        out_shape=[
            jax.ShapeDtypeStruct((B_TOTAL, 2, 14, 28), jnp.float32),
            jax.ShapeDtypeStruct((N_TILES, 1, 1), jnp.float32),
            jax.ShapeDtypeStruct((N_TILES, 1, 10), jnp.float32),
            jax.ShapeDtypeStruct((B_TOTAL, 10), jnp.float32),
        ],
        compiler_params=pltpu.CompilerParams(
            dimension_semantics=("parallel",)),
    )(x3, m1, b1t, m2, b2t, embt, m3, b3t, m4, dec2_b[None, :])

    loss, perp = pl.pallas_call(
        _fin_kernel,
        out_shape=[jax.ShapeDtypeStruct((1, 1), jnp.float32),
                   jax.ShapeDtypeStruct((1, 1), jnp.float32)],
    )(lp, cp)

    xrec = xr.transpose(0, 2, 1, 3).reshape(B_TOTAL, 1, 28, 28)
    return (xrec, loss[0, 0], perp[0, 0], enc)
